# baseline (speedup 1.0000x reference)
import jax
import jax.numpy as jnp
from jax.experimental import pallas as pl

_NUM_FEATURES = 256
_NUM_BATCHES = 8
_N_POINTS = 65536
_EPS = 1e-05
_R = 2048
_NBLK = _N_POINTS // _R


def _stats_body(x_ref, idx_ref, sums_ref, sumsq_ref, cnt_ref):
    i = pl.program_id(0)
    xb = x_ref[...]
    idx = idx_ref[0]
    iota = jax.lax.broadcasted_iota(jnp.int32, (_NUM_BATCHES, _R), 0)
    onehot = (iota == idx).astype(jnp.float32)
    dn = (((1,), (0,)), ((), ()))
    s = jax.lax.dot_general(onehot, xb, dn, preferred_element_type=jnp.float32)
    sq = jax.lax.dot_general(onehot, xb * xb, dn,
                             preferred_element_type=jnp.float32)
    c = jnp.sum(onehot, axis=1, keepdims=True)
    c = jnp.broadcast_to(c, (_NUM_BATCHES, 128))

    @pl.when(i == 0)
    def _init():
        sums_ref[...] = s
        sumsq_ref[...] = sq
        cnt_ref[...] = c

    @pl.when(i != 0)
    def _acc():
        sums_ref[...] += s
        sumsq_ref[...] += sq
        cnt_ref[...] += c


def _norm_body(x_ref, idx_ref, w_ref, b_ref, mw_ref, vw_ref,
               sums_ref, sumsq_ref, cnt_ref, o_ref):
    cnt = cnt_ref[:, 0:1]
    cs = jnp.maximum(cnt, 1.0)
    sums = sums_ref[...]
    sumsq = sumsq_ref[...]
    mean_in = sums / cs
    ex2 = sumsq / cs
    var_in = ex2 - mean_in * mean_in
    mean_ln = jnp.mean(mean_in, axis=1, keepdims=True)
    var_ln = jnp.mean(ex2, axis=1, keepdims=True) - mean_ln * mean_ln
    tot_s = jnp.sum(sums, axis=0, keepdims=True)
    tot_sq = jnp.sum(sumsq, axis=0, keepdims=True)
    n = jnp.float32(_N_POINTS)
    mean_bn = tot_s / n
    var_bn = (tot_sq - n * mean_bn * mean_bn) / (n - 1.0)

    mw = mw_ref[...]
    mw = jnp.exp(mw - jnp.max(mw, axis=1, keepdims=True))
    mw = mw / jnp.sum(mw, axis=1, keepdims=True)
    vw = vw_ref[...]
    vw = jnp.exp(vw - jnp.max(vw, axis=1, keepdims=True))
    vw = vw / jnp.sum(vw, axis=1, keepdims=True)

    mean = mw[:, 0:1] * mean_in + mw[:, 1:2] * mean_ln + mw[:, 2:3] * mean_bn
    var = vw[:, 0:1] * var_in + vw[:, 1:2] * var_ln + vw[:, 2:3] * var_bn
    inv = jax.lax.rsqrt(var + _EPS)
    scale = inv * w_ref[...]
    shift = b_ref[...] - mean * scale

    idx = idx_ref[0]
    iota = jax.lax.broadcasted_iota(jnp.int32, (_NUM_BATCHES, _R), 0)
    onehot = (iota == idx).astype(jnp.float32)
    dn = (((0,), (0,)), ((), ()))
    g_scale = jax.lax.dot_general(onehot, scale, dn,
                                  precision=jax.lax.Precision.HIGHEST,
                                  preferred_element_type=jnp.float32)
    g_shift = jax.lax.dot_general(onehot, shift, dn,
                                  precision=jax.lax.Precision.HIGHEST,
                                  preferred_element_type=jnp.float32)
    o_ref[...] = x_ref[...] * g_scale + g_shift


def kernel(x, weight, bias, mean_weight, var_weight, batch_indices):
    idx3 = batch_indices.reshape(_NBLK, 1, _R)
    mw2 = mean_weight.reshape(1, 3)
    vw2 = var_weight.reshape(1, 3)

    x_spec = pl.BlockSpec((_R, _NUM_FEATURES), lambda i: (i, 0))
    idx_spec = pl.BlockSpec((1, 1, _R), lambda i: (i, 0, 0))
    full = lambda shape: pl.BlockSpec(shape, lambda i: tuple(0 for _ in shape))

    sums, sumsq, cnt = pl.pallas_call(
        _stats_body,
        grid=(_NBLK,),
        in_specs=[x_spec, idx_spec],
        out_specs=[full((_NUM_BATCHES, _NUM_FEATURES)),
                   full((_NUM_BATCHES, _NUM_FEATURES)),
                   full((_NUM_BATCHES, 128))],
        out_shape=[
            jax.ShapeDtypeStruct((_NUM_BATCHES, _NUM_FEATURES), jnp.float32),
            jax.ShapeDtypeStruct((_NUM_BATCHES, _NUM_FEATURES), jnp.float32),
            jax.ShapeDtypeStruct((_NUM_BATCHES, 128), jnp.float32),
        ],
    )(x, idx3)

    out = pl.pallas_call(
        _norm_body,
        grid=(_NBLK,),
        in_specs=[x_spec, idx_spec,
                  full((1, _NUM_FEATURES)), full((1, _NUM_FEATURES)),
                  full((1, 3)), full((1, 3)),
                  full((_NUM_BATCHES, _NUM_FEATURES)),
                  full((_NUM_BATCHES, _NUM_FEATURES)),
                  full((_NUM_BATCHES, 128))],
        out_specs=pl.BlockSpec((_R, _NUM_FEATURES), lambda i: (i, 0)),
        out_shape=jax.ShapeDtypeStruct((_N_POINTS, _NUM_FEATURES),
                                       jnp.float32),
    )(x, idx3, weight, bias, mw2, vw2, sums, sumsq, cnt)
    return out


# speedup vs baseline: 1.1771x; 1.1771x over previous
import jax
import jax.numpy as jnp
from jax.experimental import pallas as pl

_NUM_FEATURES = 256
_NUM_BATCHES = 8
_N_POINTS = 65536
_EPS = 1e-05
_R = 1024
_NBLK = _N_POINTS // _R


def _stats_body(x_ref, idx_ref, sums_ref, sumsq_ref, cnt_ref):
    i = pl.program_id(0)
    xb = x_ref[...]
    idx = idx_ref[0]
    iota = jax.lax.broadcasted_iota(jnp.int32, (_NUM_BATCHES, _R), 0)
    onehot = (iota == idx).astype(jnp.float32)
    dn = (((1,), (0,)), ((), ()))
    s = jax.lax.dot_general(onehot, xb, dn, preferred_element_type=jnp.float32)
    sq = jax.lax.dot_general(onehot, xb * xb, dn,
                             preferred_element_type=jnp.float32)
    c = jnp.sum(onehot, axis=1, keepdims=True)
    c = jnp.broadcast_to(c, (_NUM_BATCHES, 128))

    @pl.when(i == 0)
    def _init():
        sums_ref[...] = s
        sumsq_ref[...] = sq
        cnt_ref[...] = c

    @pl.when(i != 0)
    def _acc():
        sums_ref[...] += s
        sumsq_ref[...] += sq
        cnt_ref[...] += c


def _norm_body(x_ref, idx_ref, w_ref, b_ref, mw_ref, vw_ref,
               sums_ref, sumsq_ref, cnt_ref, o_ref):
    cnt = cnt_ref[:, 0:1]
    cs = jnp.maximum(cnt, 1.0)
    sums = sums_ref[...]
    sumsq = sumsq_ref[...]
    mean_in = sums / cs
    ex2 = sumsq / cs
    var_in = ex2 - mean_in * mean_in
    mean_ln = jnp.mean(mean_in, axis=1, keepdims=True)
    var_ln = jnp.mean(ex2, axis=1, keepdims=True) - mean_ln * mean_ln
    tot_s = jnp.sum(sums, axis=0, keepdims=True)
    tot_sq = jnp.sum(sumsq, axis=0, keepdims=True)
    n = jnp.float32(_N_POINTS)
    mean_bn = tot_s / n
    var_bn = (tot_sq - n * mean_bn * mean_bn) / (n - 1.0)

    mw = mw_ref[...]
    mw = jnp.exp(mw - jnp.max(mw, axis=1, keepdims=True))
    mw = mw / jnp.sum(mw, axis=1, keepdims=True)
    vw = vw_ref[...]
    vw = jnp.exp(vw - jnp.max(vw, axis=1, keepdims=True))
    vw = vw / jnp.sum(vw, axis=1, keepdims=True)

    mean = mw[:, 0:1] * mean_in + mw[:, 1:2] * mean_ln + mw[:, 2:3] * mean_bn
    var = vw[:, 0:1] * var_in + vw[:, 1:2] * var_ln + vw[:, 2:3] * var_bn
    inv = jax.lax.rsqrt(var + _EPS)
    scale = inv * w_ref[...]
    shift = b_ref[...] - mean * scale

    idx = idx_ref[0]
    iota = jax.lax.broadcasted_iota(jnp.int32, (_NUM_BATCHES, _R), 0)
    onehot = (iota == idx).astype(jnp.float32)
    dn = (((0,), (0,)), ((), ()))
    g_scale = jax.lax.dot_general(onehot, scale, dn,
                                  preferred_element_type=jnp.float32)
    g_shift = jax.lax.dot_general(onehot, shift, dn,
                                  preferred_element_type=jnp.float32)
    o_ref[...] = x_ref[...] * g_scale + g_shift


def kernel(x, weight, bias, mean_weight, var_weight, batch_indices):
    idx3 = batch_indices.reshape(_NBLK, 1, _R)
    mw2 = mean_weight.reshape(1, 3)
    vw2 = var_weight.reshape(1, 3)

    x_spec = pl.BlockSpec((_R, _NUM_FEATURES), lambda i: (i, 0))
    idx_spec = pl.BlockSpec((1, 1, _R), lambda i: (i, 0, 0))
    full = lambda shape: pl.BlockSpec(shape, lambda i: tuple(0 for _ in shape))

    sums, sumsq, cnt = pl.pallas_call(
        _stats_body,
        grid=(_NBLK,),
        in_specs=[x_spec, idx_spec],
        out_specs=[full((_NUM_BATCHES, _NUM_FEATURES)),
                   full((_NUM_BATCHES, _NUM_FEATURES)),
                   full((_NUM_BATCHES, 128))],
        out_shape=[
            jax.ShapeDtypeStruct((_NUM_BATCHES, _NUM_FEATURES), jnp.float32),
            jax.ShapeDtypeStruct((_NUM_BATCHES, _NUM_FEATURES), jnp.float32),
            jax.ShapeDtypeStruct((_NUM_BATCHES, 128), jnp.float32),
        ],
    )(x, idx3)

    out = pl.pallas_call(
        _norm_body,
        grid=(_NBLK,),
        in_specs=[x_spec, idx_spec,
                  full((1, _NUM_FEATURES)), full((1, _NUM_FEATURES)),
                  full((1, 3)), full((1, 3)),
                  full((_NUM_BATCHES, _NUM_FEATURES)),
                  full((_NUM_BATCHES, _NUM_FEATURES)),
                  full((_NUM_BATCHES, 128))],
        out_specs=pl.BlockSpec((_R, _NUM_FEATURES), lambda i: (i, 0)),
        out_shape=jax.ShapeDtypeStruct((_N_POINTS, _NUM_FEATURES),
                                       jnp.float32),
    )(x, idx3, weight, bias, mw2, vw2, sums, sumsq, cnt)
    return out


# speedup vs baseline: 2.1509x; 1.8272x over previous
import jax
import jax.numpy as jnp
from jax.experimental import pallas as pl

_NUM_FEATURES = 256
_NUM_BATCHES = 8
_N_POINTS = 65536
_EPS = 1e-05
_R = 4096
_NBLK = _N_POINTS // _R


def _stats_body(x_ref, idx_ref, sums_ref, sumsq_ref, cnt_ref):
    i = pl.program_id(0)
    xb = x_ref[...]
    idx = idx_ref[0]
    iota = jax.lax.broadcasted_iota(jnp.int32, (_NUM_BATCHES, _R), 0)
    onehot = (iota == idx).astype(jnp.float32)
    dn = (((1,), (0,)), ((), ()))
    s = jax.lax.dot_general(onehot, xb, dn, preferred_element_type=jnp.float32)
    sq = jax.lax.dot_general(onehot, xb * xb, dn,
                             preferred_element_type=jnp.float32)
    c = jnp.sum(onehot, axis=1, keepdims=True)
    c = jnp.broadcast_to(c, (_NUM_BATCHES, 128))

    @pl.when(i == 0)
    def _init():
        sums_ref[...] = s
        sumsq_ref[...] = sq
        cnt_ref[...] = c

    @pl.when(i != 0)
    def _acc():
        sums_ref[...] += s
        sumsq_ref[...] += sq
        cnt_ref[...] += c


def _norm_body(x_ref, idx_ref, w_ref, b_ref, mw_ref, vw_ref,
               sums_ref, sumsq_ref, cnt_ref, o_ref):
    cnt = cnt_ref[:, 0:1]
    cs = jnp.maximum(cnt, 1.0)
    sums = sums_ref[...]
    sumsq = sumsq_ref[...]
    mean_in = sums / cs
    ex2 = sumsq / cs
    var_in = ex2 - mean_in * mean_in
    mean_ln = jnp.mean(mean_in, axis=1, keepdims=True)
    var_ln = jnp.mean(ex2, axis=1, keepdims=True) - mean_ln * mean_ln
    tot_s = jnp.sum(sums, axis=0, keepdims=True)
    tot_sq = jnp.sum(sumsq, axis=0, keepdims=True)
    n = jnp.float32(_N_POINTS)
    mean_bn = tot_s / n
    var_bn = (tot_sq - n * mean_bn * mean_bn) / (n - 1.0)

    mw = mw_ref[...]
    mw = jnp.exp(mw - jnp.max(mw, axis=1, keepdims=True))
    mw = mw / jnp.sum(mw, axis=1, keepdims=True)
    vw = vw_ref[...]
    vw = jnp.exp(vw - jnp.max(vw, axis=1, keepdims=True))
    vw = vw / jnp.sum(vw, axis=1, keepdims=True)

    mean = mw[:, 0:1] * mean_in + mw[:, 1:2] * mean_ln + mw[:, 2:3] * mean_bn
    var = vw[:, 0:1] * var_in + vw[:, 1:2] * var_ln + vw[:, 2:3] * var_bn
    inv = jax.lax.rsqrt(var + _EPS)
    scale = inv * w_ref[...]
    shift = b_ref[...] - mean * scale

    idx = idx_ref[0]
    iota = jax.lax.broadcasted_iota(jnp.int32, (_NUM_BATCHES, _R), 0)
    onehot = (iota == idx).astype(jnp.float32)
    dn = (((0,), (0,)), ((), ()))
    g_scale = jax.lax.dot_general(onehot, scale, dn,
                                  preferred_element_type=jnp.float32)
    g_shift = jax.lax.dot_general(onehot, shift, dn,
                                  preferred_element_type=jnp.float32)
    o_ref[...] = x_ref[...] * g_scale + g_shift


def kernel(x, weight, bias, mean_weight, var_weight, batch_indices):
    idx3 = batch_indices.reshape(_NBLK, 1, _R)
    mw2 = mean_weight.reshape(1, 3)
    vw2 = var_weight.reshape(1, 3)

    x_spec = pl.BlockSpec((_R, _NUM_FEATURES), lambda i: (i, 0))
    idx_spec = pl.BlockSpec((1, 1, _R), lambda i: (i, 0, 0))
    full = lambda shape: pl.BlockSpec(shape, lambda i: tuple(0 for _ in shape))

    sums, sumsq, cnt = pl.pallas_call(
        _stats_body,
        grid=(_NBLK,),
        in_specs=[x_spec, idx_spec],
        out_specs=[full((_NUM_BATCHES, _NUM_FEATURES)),
                   full((_NUM_BATCHES, _NUM_FEATURES)),
                   full((_NUM_BATCHES, 128))],
        out_shape=[
            jax.ShapeDtypeStruct((_NUM_BATCHES, _NUM_FEATURES), jnp.float32),
            jax.ShapeDtypeStruct((_NUM_BATCHES, _NUM_FEATURES), jnp.float32),
            jax.ShapeDtypeStruct((_NUM_BATCHES, 128), jnp.float32),
        ],
    )(x, idx3)

    out = pl.pallas_call(
        _norm_body,
        grid=(_NBLK,),
        in_specs=[x_spec, idx_spec,
                  full((1, _NUM_FEATURES)), full((1, _NUM_FEATURES)),
                  full((1, 3)), full((1, 3)),
                  full((_NUM_BATCHES, _NUM_FEATURES)),
                  full((_NUM_BATCHES, _NUM_FEATURES)),
                  full((_NUM_BATCHES, 128))],
        out_specs=pl.BlockSpec((_R, _NUM_FEATURES), lambda i: (i, 0)),
        out_shape=jax.ShapeDtypeStruct((_N_POINTS, _NUM_FEATURES),
                                       jnp.float32),
    )(x, idx3, weight, bias, mw2, vw2, sums, sumsq, cnt)
    return out


# speedup vs baseline: 2.3180x; 1.0777x over previous
import jax
import jax.numpy as jnp
from jax.experimental import pallas as pl

_NUM_FEATURES = 256
_NUM_BATCHES = 8
_N_POINTS = 65536
_EPS = 1e-05
_R = 8192
_NBLK = _N_POINTS // _R


def _stats_body(x_ref, idx_ref, sums_ref, sumsq_ref, cnt_ref):
    i = pl.program_id(0)
    xb = x_ref[...]
    idx = idx_ref[0]
    iota = jax.lax.broadcasted_iota(jnp.int32, (_NUM_BATCHES, _R), 0)
    onehot = (iota == idx).astype(jnp.float32)
    dn = (((1,), (0,)), ((), ()))
    s = jax.lax.dot_general(onehot, xb, dn, preferred_element_type=jnp.float32)
    sq = jax.lax.dot_general(onehot, xb * xb, dn,
                             preferred_element_type=jnp.float32)
    c = jnp.sum(onehot, axis=1, keepdims=True)
    c = jnp.broadcast_to(c, (_NUM_BATCHES, 128))

    @pl.when(i == 0)
    def _init():
        sums_ref[...] = s
        sumsq_ref[...] = sq
        cnt_ref[...] = c

    @pl.when(i != 0)
    def _acc():
        sums_ref[...] += s
        sumsq_ref[...] += sq
        cnt_ref[...] += c


def _norm_body(x_ref, idx_ref, w_ref, b_ref, mw_ref, vw_ref,
               sums_ref, sumsq_ref, cnt_ref, o_ref):
    cnt = cnt_ref[:, 0:1]
    cs = jnp.maximum(cnt, 1.0)
    sums = sums_ref[...]
    sumsq = sumsq_ref[...]
    mean_in = sums / cs
    ex2 = sumsq / cs
    var_in = ex2 - mean_in * mean_in
    mean_ln = jnp.mean(mean_in, axis=1, keepdims=True)
    var_ln = jnp.mean(ex2, axis=1, keepdims=True) - mean_ln * mean_ln
    tot_s = jnp.sum(sums, axis=0, keepdims=True)
    tot_sq = jnp.sum(sumsq, axis=0, keepdims=True)
    n = jnp.float32(_N_POINTS)
    mean_bn = tot_s / n
    var_bn = (tot_sq - n * mean_bn * mean_bn) / (n - 1.0)

    mw = mw_ref[...]
    mw = jnp.exp(mw - jnp.max(mw, axis=1, keepdims=True))
    mw = mw / jnp.sum(mw, axis=1, keepdims=True)
    vw = vw_ref[...]
    vw = jnp.exp(vw - jnp.max(vw, axis=1, keepdims=True))
    vw = vw / jnp.sum(vw, axis=1, keepdims=True)

    mean = mw[:, 0:1] * mean_in + mw[:, 1:2] * mean_ln + mw[:, 2:3] * mean_bn
    var = vw[:, 0:1] * var_in + vw[:, 1:2] * var_ln + vw[:, 2:3] * var_bn
    inv = jax.lax.rsqrt(var + _EPS)
    scale = inv * w_ref[...]
    shift = b_ref[...] - mean * scale

    idx = idx_ref[0]
    iota = jax.lax.broadcasted_iota(jnp.int32, (_NUM_BATCHES, _R), 0)
    onehot = (iota == idx).astype(jnp.float32)
    dn = (((0,), (0,)), ((), ()))
    g_scale = jax.lax.dot_general(onehot, scale, dn,
                                  preferred_element_type=jnp.float32)
    g_shift = jax.lax.dot_general(onehot, shift, dn,
                                  preferred_element_type=jnp.float32)
    o_ref[...] = x_ref[...] * g_scale + g_shift


def kernel(x, weight, bias, mean_weight, var_weight, batch_indices):
    idx3 = batch_indices.reshape(_NBLK, 1, _R)
    mw2 = mean_weight.reshape(1, 3)
    vw2 = var_weight.reshape(1, 3)

    x_spec = pl.BlockSpec((_R, _NUM_FEATURES), lambda i: (i, 0))
    idx_spec = pl.BlockSpec((1, 1, _R), lambda i: (i, 0, 0))
    full = lambda shape: pl.BlockSpec(shape, lambda i: tuple(0 for _ in shape))

    sums, sumsq, cnt = pl.pallas_call(
        _stats_body,
        grid=(_NBLK,),
        in_specs=[x_spec, idx_spec],
        out_specs=[full((_NUM_BATCHES, _NUM_FEATURES)),
                   full((_NUM_BATCHES, _NUM_FEATURES)),
                   full((_NUM_BATCHES, 128))],
        out_shape=[
            jax.ShapeDtypeStruct((_NUM_BATCHES, _NUM_FEATURES), jnp.float32),
            jax.ShapeDtypeStruct((_NUM_BATCHES, _NUM_FEATURES), jnp.float32),
            jax.ShapeDtypeStruct((_NUM_BATCHES, 128), jnp.float32),
        ],
    )(x, idx3)

    out = pl.pallas_call(
        _norm_body,
        grid=(_NBLK,),
        in_specs=[x_spec, idx_spec,
                  full((1, _NUM_FEATURES)), full((1, _NUM_FEATURES)),
                  full((1, 3)), full((1, 3)),
                  full((_NUM_BATCHES, _NUM_FEATURES)),
                  full((_NUM_BATCHES, _NUM_FEATURES)),
                  full((_NUM_BATCHES, 128))],
        out_specs=pl.BlockSpec((_R, _NUM_FEATURES), lambda i: (i, 0)),
        out_shape=jax.ShapeDtypeStruct((_N_POINTS, _NUM_FEATURES),
                                       jnp.float32),
    )(x, idx3, weight, bias, mw2, vw2, sums, sumsq, cnt)
    return out
